# baseline (speedup 1.0000x reference)
import functools

import jax
import jax.numpy as jnp
from jax import lax
from jax.experimental import pallas as pl
from jax.experimental.pallas import tpu as pltpu
from jax.experimental.pallas import tpu_sc as plsc

_NUM_WORKERS = 32
_CHUNK = 128
_LANES = 16


def _emb_lookup(idx_flat, token_table, pos_table, n_rows, L, D):
    rows_per_w = n_rows // _NUM_WORKERS
    n_chunks = rows_per_w // _CHUNK

    mesh = plsc.VectorSubcoreMesh(core_axis_name="c", subcore_axis_name="s")

    @functools.partial(
        pl.kernel,
        out_type=jax.ShapeDtypeStruct((n_rows, D), jnp.float32),
        mesh=mesh,
        scratch_types=[
            pltpu.VMEM((rows_per_w,), jnp.int32),
            pltpu.VMEM((2 * L, D), jnp.float32),
            pltpu.VMEM((_CHUNK, D), jnp.float32),
        ],
        compiler_params=pltpu.CompilerParams(use_tc_tiling_on_sc=False),
    )
    def emb(idx_hbm, tok_hbm, pos_hbm, out_hbm, idx_v, pos_v, buf_v):
        wid = lax.axis_index("s") * 2 + lax.axis_index("c")
        base = wid * rows_per_w
        pltpu.sync_copy(idx_hbm.at[pl.ds(base, rows_per_w)], idx_v)
        pltpu.sync_copy(pos_hbm, pos_v.at[pl.ds(0, L)])
        pltpu.sync_copy(pos_hbm, pos_v.at[pl.ds(L, L)])

        @pl.loop(0, n_chunks)
        def _chunk_body(c):
            r0 = c * _CHUNK
            pltpu.sync_copy(tok_hbm.at[idx_v.at[pl.ds(r0, _CHUNK)]], buf_v)
            pos_off = lax.rem(r0, L)

            @pl.loop(0, _CHUNK)
            def _row_body(r):
                for d in range(D // _LANES):
                    slc = pl.ds(d * _LANES, _LANES)
                    buf_v[r, slc] = buf_v[r, slc] + pos_v[pos_off + r, slc]

            pltpu.sync_copy(buf_v, out_hbm.at[pl.ds(base + r0, _CHUNK)])

    return emb(idx_flat, token_table, pos_table)


def kernel(inputs, token_table, pos_table):
    B, L = inputs.shape
    V, D = token_table.shape
    n_rows = B * L
    idx_flat = inputs.reshape(n_rows).astype(jnp.int32)
    out = _emb_lookup(idx_flat, token_table, pos_table, n_rows, L, D)
    return out.reshape(B, L, D)


# speedup vs baseline: 1.4009x; 1.4009x over previous
import functools

import jax
import jax.numpy as jnp
from jax import lax
from jax.experimental import pallas as pl
from jax.experimental.pallas import tpu as pltpu
from jax.experimental.pallas import tpu_sc as plsc

_NUM_WORKERS = 32
_LANES = 16
_NBUF = 4
_G1 = 128


def _emb_lookup(idx_flat, token_table, pos_table, n_rows, L, D):
    rows_per_w = n_rows // _NUM_WORKERS
    n_sb = rows_per_w // L
    g2 = L - _G1

    mesh = plsc.VectorSubcoreMesh(core_axis_name="c", subcore_axis_name="s")

    scratch = [
        pltpu.VMEM((rows_per_w,), jnp.int32),
        pltpu.VMEM((L, D), jnp.float32),
    ]
    scratch += [pltpu.VMEM((L, D), jnp.float32) for _ in range(_NBUF)]
    scratch += [pltpu.SemaphoreType.DMA for _ in range(2 * _NBUF)]

    @functools.partial(
        pl.kernel,
        out_type=jax.ShapeDtypeStruct((n_rows, D), jnp.float32),
        mesh=mesh,
        scratch_types=scratch,
        compiler_params=pltpu.CompilerParams(use_tc_tiling_on_sc=False),
    )
    def emb(idx_hbm, tok_hbm, pos_hbm, out_hbm, idx_v, pos_v, *rest):
        bufs = rest[:_NBUF]
        gsems = rest[_NBUF:2 * _NBUF]
        ssems = rest[2 * _NBUF:]
        wid = lax.axis_index("s") * 2 + lax.axis_index("c")
        base = wid * rows_per_w
        pltpu.sync_copy(idx_hbm.at[pl.ds(base, rows_per_w)], idx_v)
        pltpu.sync_copy(pos_hbm, pos_v)

        def start_gathers(sb, k):
            r0 = sb * L
            h1 = pltpu.async_copy(
                tok_hbm.at[idx_v.at[pl.ds(r0, _G1)]],
                bufs[k].at[pl.ds(0, _G1)], gsems[k])
            h2 = pltpu.async_copy(
                tok_hbm.at[idx_v.at[pl.ds(r0 + _G1, g2)]],
                bufs[k].at[pl.ds(_G1, g2)], gsems[k])
            return h1, h2

        @pl.loop(0, n_sb, step=_NBUF)
        def _sb_body(sb):
            handles = [start_gathers(sb + k, k) for k in range(_NBUF)]
            for k in range(_NBUF):
                cur = sb + k
                for h in handles[k]:
                    h.wait()

                @pl.loop(0, L, step=8)
                def _row_body(r, _k=k):
                    for rr in range(8):
                        for d in range(D // _LANES):
                            slc = pl.ds(d * _LANES, _LANES)
                            bufs[_k][r + rr, slc] = (
                                bufs[_k][r + rr, slc] + pos_v[r + rr, slc])

                pltpu.sync_copy(bufs[k], out_hbm.at[pl.ds(base + cur * L, L)])

    return emb(idx_flat, token_table, pos_table)


def kernel(inputs, token_table, pos_table):
    B, L = inputs.shape
    V, D = token_table.shape
    n_rows = B * L
    idx_flat = inputs.reshape(n_rows).astype(jnp.int32)
    out = _emb_lookup(idx_flat, token_table, pos_table, n_rows, L, D)
    return out.reshape(B, L, D)


# speedup vs baseline: 1.4027x; 1.0013x over previous
import functools

import jax
import jax.numpy as jnp
from jax import lax
from jax.experimental import pallas as pl
from jax.experimental.pallas import tpu as pltpu
from jax.experimental.pallas import tpu_sc as plsc

_NUM_WORKERS = 32
_LANES = 16
_NBUF = 4
_G1 = 128


def _emb_lookup(ids, token_table, pos_table):
    B, L = ids.shape
    V, D = token_table.shape
    seq_per_w = B // _NUM_WORKERS
    g2 = L - _G1

    mesh = plsc.VectorSubcoreMesh(core_axis_name="c", subcore_axis_name="s")

    scratch = [
        pltpu.VMEM((seq_per_w, L), jnp.int32),
        pltpu.VMEM((L, D), jnp.float32),
    ]
    scratch += [pltpu.VMEM((L, D), jnp.float32) for _ in range(_NBUF)]
    scratch += [pltpu.SemaphoreType.DMA for _ in range(_NBUF)]

    @functools.partial(
        pl.kernel,
        out_type=jax.ShapeDtypeStruct((B, L, D), jnp.float32),
        mesh=mesh,
        scratch_types=scratch,
        compiler_params=pltpu.CompilerParams(use_tc_tiling_on_sc=False),
    )
    def emb(ids_hbm, tok_hbm, pos_hbm, out_hbm, idx_v, pos_v, *rest):
        bufs = rest[:_NBUF]
        gsems = rest[_NBUF:]
        wid = lax.axis_index("s") * 2 + lax.axis_index("c")
        sbase = wid * seq_per_w
        pltpu.sync_copy(ids_hbm.at[pl.ds(sbase, seq_per_w)], idx_v)
        pltpu.sync_copy(pos_hbm, pos_v)

        def start_gathers(seq, k):
            h1 = pltpu.async_copy(
                tok_hbm.at[idx_v.at[seq, pl.ds(0, _G1)]],
                bufs[k].at[pl.ds(0, _G1)], gsems[k])
            h2 = pltpu.async_copy(
                tok_hbm.at[idx_v.at[seq, pl.ds(_G1, g2)]],
                bufs[k].at[pl.ds(_G1, g2)], gsems[k])
            return h1, h2

        @pl.loop(0, seq_per_w, step=_NBUF)
        def _seq_body(seq):
            handles = [start_gathers(seq + k, k) for k in range(_NBUF)]
            for k in range(_NBUF):
                for h in handles[k]:
                    h.wait()

                @pl.loop(0, L, step=8)
                def _row_body(r, _k=k):
                    for rr in range(8):
                        for d in range(D // _LANES):
                            slc = pl.ds(d * _LANES, _LANES)
                            bufs[_k][r + rr, slc] = (
                                bufs[_k][r + rr, slc] + pos_v[r + rr, slc])

                pltpu.sync_copy(bufs[k], out_hbm.at[sbase + seq + k])

    return emb(ids, token_table, pos_table)


def kernel(inputs, token_table, pos_table):
    return _emb_lookup(inputs.astype(jnp.int32), token_table, pos_table)


# speedup vs baseline: 1.5389x; 1.0971x over previous
import functools

import jax
import jax.numpy as jnp
from jax import lax
from jax.experimental import pallas as pl
from jax.experimental.pallas import tpu as pltpu
from jax.experimental.pallas import tpu_sc as plsc

_NUM_WORKERS = 32
_NBUF = 4
_G1 = 128
_TBLK = 1024


def _repack_table(token_table_t, V, D):
    n_in = (V + _TBLK - 1) // _TBLK
    grid = (n_in + 1) // 2
    out_rows = grid * _TBLK

    def body(lo_ref, hi_ref, o_ref):
        o_ref[:, 0:D] = lo_ref[...].T
        o_ref[:, D:2 * D] = hi_ref[...].T

    packed = pl.pallas_call(
        body,
        out_shape=jax.ShapeDtypeStruct((out_rows, 2 * D), jnp.float32),
        grid=(grid,),
        in_specs=[
            pl.BlockSpec((D, _TBLK), lambda b: (0, 2 * b)),
            pl.BlockSpec(
                (D, _TBLK),
                lambda b, m=n_in - 1: (0, jnp.minimum(2 * b + 1, m))),
        ],
        out_specs=pl.BlockSpec((_TBLK, 2 * D), lambda b: (b, 0)),
    )(token_table_t, token_table_t)
    return packed, out_rows


def _sc_gather(ids, tok_lin, B, L, D):
    n_rows = B * L
    rows_per_w = n_rows // _NUM_WORKERS
    seq_per_w = rows_per_w // L
    g2 = L - _G1

    mesh = plsc.VectorSubcoreMesh(core_axis_name="c", subcore_axis_name="s")

    scratch = [pltpu.VMEM((rows_per_w,), jnp.int32)]
    scratch += [pltpu.VMEM((L, D), jnp.float32) for _ in range(_NBUF)]
    scratch += [pltpu.SemaphoreType.DMA for _ in range(_NBUF)]

    @functools.partial(
        pl.kernel,
        out_type=jax.ShapeDtypeStruct((n_rows, D), jnp.float32),
        mesh=mesh,
        scratch_types=scratch,
        compiler_params=pltpu.CompilerParams(use_tc_tiling_on_sc=False),
    )
    def emb(ids_hbm, tok_hbm, out_hbm, idx_v, *rest):
        bufs = rest[:_NBUF]
        gsems = rest[_NBUF:]
        wid = lax.axis_index("s") * 2 + lax.axis_index("c")
        base = wid * rows_per_w
        pltpu.sync_copy(ids_hbm.at[pl.ds(base, rows_per_w)], idx_v)

        def start_seq(seq, k):
            r0 = seq * L
            h1 = pltpu.async_copy(
                tok_hbm.at[idx_v.at[pl.ds(r0, _G1)]],
                bufs[k].at[pl.ds(0, _G1)], gsems[k])
            h2 = pltpu.async_copy(
                tok_hbm.at[idx_v.at[pl.ds(r0 + _G1, g2)]],
                bufs[k].at[pl.ds(_G1, g2)], gsems[k])
            return h1, h2

        @pl.loop(0, seq_per_w, step=_NBUF)
        def _seq_body(seq):
            handles = [start_seq(seq + k, k) for k in range(_NBUF)]
            for k in range(_NBUF):
                for h in handles[k]:
                    h.wait()
                pltpu.sync_copy(
                    bufs[k], out_hbm.at[pl.ds(base + (seq + k) * L, L)])

    return emb(ids, tok_lin)


def kernel(inputs, token_table, pos_table):
    B, L = inputs.shape
    V, D = token_table.shape
    ids = inputs.astype(jnp.int32).reshape(B * L)
    tok_packed, out_rows = _repack_table(token_table.T, V, D)
    tok_lin = tok_packed.reshape(2 * out_rows, D)
    q, rem = ids // (2 * _TBLK), ids % (2 * _TBLK)
    ids_m = jnp.where(
        rem < _TBLK,
        2 * (_TBLK * q + rem),
        2 * (_TBLK * q + rem - _TBLK) + 1)
    gathered = _sc_gather(ids_m, tok_lin, B, L, D)
    return gathered.reshape(B, L, D) + pos_table[None, :, :]
